# baseline (speedup 1.0000x reference)
import jax
import jax.numpy as jnp
from jax.experimental import pallas as pl
from jax.experimental.pallas import tpu as pltpu

_ROWS = 4096
_COLS = 4096
_BLOCK = 512


def _body(x_ref, out_ref):
    i = pl.program_id(0)
    m = (x_ref[...] != 0.0).astype(jnp.int32)
    row_ids = jax.lax.broadcasted_iota(jnp.int32, (_BLOCK, _COLS), 0) + i * _BLOCK
    col_ids = jax.lax.broadcasted_iota(jnp.int32, (_BLOCK, _COLS), 1)
    p_row = jnp.sum(row_ids * m)
    p_col = jnp.sum(col_ids * m)

    @pl.when(i == 0)
    def _init():
        out_ref[0] = 0
        out_ref[1] = 0

    out_ref[0] += p_row
    out_ref[1] += p_col


def kernel(inputs):
    return pl.pallas_call(
        _body,
        grid=(_ROWS // _BLOCK,),
        in_specs=[pl.BlockSpec((_BLOCK, _COLS), lambda i: (i, 0))],
        out_specs=pl.BlockSpec(memory_space=pltpu.SMEM),
        out_shape=jax.ShapeDtypeStruct((2,), jnp.int32),
    )(inputs)


# speedup vs baseline: 1.1906x; 1.1906x over previous
import jax
import jax.numpy as jnp
from jax.experimental import pallas as pl
from jax.experimental.pallas import tpu as pltpu

_ROWS = 4096
_COLS = 4096
_BLOCK = 512


def _body(x_ref, out_ref):
    i = pl.program_id(0)
    m = (x_ref[...] != 0.0).astype(jnp.float32)
    sel = jax.lax.broadcasted_iota(jnp.int32, (2, _BLOCK), 0) == 0
    lane = jax.lax.broadcasted_iota(jnp.int32, (2, _BLOCK), 1).astype(jnp.float32)
    w = jnp.where(sel, lane, 1.0)
    r = jax.lax.dot_general(w, m, (((1,), (0,)), ((), ())),
                            preferred_element_type=jnp.float32)
    ri = r.astype(jnp.int32)
    col_ids = jax.lax.broadcasted_iota(jnp.int32, (1, _COLS), 1)
    s_local = jnp.sum(ri[0:1])
    nnz = jnp.sum(ri[1:2])
    p_row = s_local + (i * _BLOCK) * nnz
    p_col = jnp.sum(ri[1:2] * col_ids)

    @pl.when(i == 0)
    def _init():
        out_ref[0] = 0
        out_ref[1] = 0

    out_ref[0] += p_row
    out_ref[1] += p_col


def kernel(inputs):
    return pl.pallas_call(
        _body,
        grid=(_ROWS // _BLOCK,),
        in_specs=[pl.BlockSpec((_BLOCK, _COLS), lambda i: (i, 0))],
        out_specs=pl.BlockSpec(memory_space=pltpu.SMEM),
        out_shape=jax.ShapeDtypeStruct((2,), jnp.int32),
    )(inputs)
